# Initial kernel scaffold; baseline (speedup 1.0000x reference)
import jax
import jax.numpy as jnp
from jax.experimental import pallas as pl


def kernel(x, pos_table):
    raise NotImplementedError("write your pallas kernel here")


# speedup vs baseline: 1.6940x; 1.6940x over previous
import jax
import jax.numpy as jnp
from jax.experimental import pallas as pl


_TL = 512


def _add_body(x_ref, pe_ref, o_ref):
    o_ref[...] = x_ref[...] + pe_ref[...]


def kernel(x, pos_table):
    B, L, D = x.shape
    nblk = L // _TL
    return pl.pallas_call(
        _add_body,
        grid=(nblk, B),
        in_specs=[
            pl.BlockSpec((1, _TL, D), lambda l, b: (b, l, 0)),
            pl.BlockSpec((_TL, D), lambda l, b: (l, 0)),
        ],
        out_specs=pl.BlockSpec((1, _TL, D), lambda l, b: (b, l, 0)),
        out_shape=jax.ShapeDtypeStruct((B, L, D), x.dtype),
    )(x, pos_table)
